# Initial kernel scaffold; baseline (speedup 1.0000x reference)
import jax
import jax.numpy as jnp
from jax.experimental import pallas as pl


def kernel(x, edge_index, node_idx, labels, p_hol, p_shared, combine_weight, p_balance, W1, W2, alpha):
    raise NotImplementedError("write your pallas kernel here")


# speedup vs baseline: 1.5522x; 1.5522x over previous
import functools

import jax
import jax.numpy as jnp
from jax import lax
from jax.experimental import pallas as pl
from jax.experimental.pallas import tpu as pltpu
from jax.experimental.pallas import tpu_sc as plsc

N0 = 10000
D = 128
E0 = 160000
H = 128
C = 64
K1 = 33
NSEL = 2048
TEMP = 0.2
EPS = 1e-8

NP = 10240
KP = 48
ND = NP - 1
EP = 163840
BLKR = 512
NBLK = NP // BLKR


def _t1_body(x_ref, q_ref, w1_ref, fea_ref, z1_ref):
    xa = x_ref[...] * q_ref[...]
    fea = jnp.where(xa > 0, xa, jnp.exp(jnp.minimum(xa, 0.0)) - 1.0)
    fea_ref[...] = fea
    z1_ref[...] = jnp.dot(fea, w1_ref[...], preferred_element_type=jnp.float32)


def _t2_body(dp_ref, fea_ref, z1_ref, dis_ref, g_ref, t1_ref):
    d = 1.0 + dp_ref[0, :, 0:1] + dp_ref[1, :, 0:1]
    dis = lax.rsqrt(d)
    dis_b = jnp.broadcast_to(dis, (BLKR, D))
    dis_ref[...] = dis_b
    g_ref[...] = dis_b * fea_ref[...]
    t1_ref[...] = dis_b * z1_ref[...]


def _t3_body(fea_ref, dis_ref, g_ref, tmp_ref, pb_ref, hn_ref):
    agg = dis_ref[...] * (tmp_ref[0] + tmp_ref[1] + g_ref[...])
    hcat = jnp.concatenate([fea_ref[...], agg], axis=1) * pb_ref[...]
    nrm = jnp.sqrt(jnp.sum(hcat * hcat, axis=1, keepdims=True))
    hn_ref[...] = hcat / (nrm + EPS)


def _t4_body(hn_blk_ref, hnt_ref, nbr_ref, nbrw_ref, vals_ref, sims_ref):
    sims = jnp.dot(hn_blk_ref[...], hnt_ref[...],
                   preferred_element_type=jnp.float32)
    colio = lax.broadcasted_iota(jnp.int32, (BLKR, NP), 1)
    sims_ref[...] = jnp.where(colio >= N0, -2.0, sims)
    kio = lax.broadcasted_iota(jnp.int32, (BLKR, KP), 1)
    kio2 = lax.broadcasted_iota(jnp.int32, (BLKR, D), 1)
    vals_acc = jnp.zeros((BLKR, KP), jnp.float32)
    nbr_acc = jnp.full((BLKR, KP), ND, jnp.int32)
    nbrw_acc = jnp.full((BLKR, D), ND, jnp.int32)
    for k in range(K1):
        s = sims_ref[...]
        m = jnp.max(s, axis=1, keepdims=True)
        idx = jnp.min(jnp.where(s == m, colio, 2 ** 30), axis=1, keepdims=True)
        vals_acc = jnp.where(kio == k, m, vals_acc)
        nbr_acc = jnp.where(kio == k, idx, nbr_acc)
        nbrw_acc = jnp.where(kio2 == k, idx, nbrw_acc)
        sims_ref[...] = jnp.where(colio == idx, -3.0, s)
    vals_ref[...] = vals_acc
    nbr_ref[...] = nbr_acc
    nbrw_ref[...] = nbrw_acc


def _t5_body(al_ref, s1p_ref, feat_ref, dis_ref, s2_ref, s3p_ref, w2_ref,
             z2_ref, t2_ref):
    al = al_ref[0, 0]
    dis = dis_ref[...]
    pn = dis * (s1p_ref[0] + s1p_ref[1]) + dis * dis * feat_ref[...]
    h1 = jnp.maximum(
        al * pn + (1.0 - al) * (s2_ref[...] + s3p_ref[0] + s3p_ref[1]), 0.0)
    z2 = jnp.dot(h1, w2_ref[...], preferred_element_type=jnp.float32)
    z2_ref[...] = z2
    t2_ref[...] = dis * z2


def _t6a_body(al_ref, s1p_ref, feat_ref, dis_ref, s2_ref, s3p_ref, out_ref):
    al = al_ref[0, 0]
    dis = dis_ref[...]
    pn = dis * (s1p_ref[0] + s1p_ref[1]) + dis * dis * feat_ref[...]
    out_ref[...] = al * pn + (1.0 - al) * (s2_ref[...] + s3p_ref[0] + s3p_ref[1])


def _t6b_body(out_ref, nidx_ref, lab_ref, logits_ref):
    rb = 1024
    nidx = nidx_ref[...]
    sel = jnp.zeros((NSEL, H), jnp.float32)
    for j in range(NP // rb):
        cio = lax.broadcasted_iota(jnp.int32, (NSEL, rb), 1) + j * rb
        p = (cio == nidx).astype(jnp.float32)
        sel = sel + jnp.dot(p, out_ref[pl.ds(j * rb, rb), :],
                            preferred_element_type=jnp.float32)
    lio = lax.broadcasted_iota(jnp.int32, (NSEL, C), 1)
    lmat = (lio == lab_ref[...]).astype(jnp.float32)
    sums = lax.dot_general(lmat, sel, (((0,), (0,)), ((), ())),
                           preferred_element_type=jnp.float32)
    cnts = lax.dot_general(lmat, jnp.ones((NSEL, 1), jnp.float32),
                           (((0,), (0,)), ((), ())),
                           preferred_element_type=jnp.float32)
    proto = sums / jnp.maximum(cnts, 1.0)
    na = jnp.sqrt(jnp.sum(sel * sel, axis=1, keepdims=True))
    a = sel / (na + EPS)
    nb = jnp.sqrt(jnp.sum(proto * proto, axis=1, keepdims=True))
    b = proto / (nb + EPS)
    logits_ref[...] = lax.dot_general(
        a, b, (((1,), (1,)), ((), ())),
        preferred_element_type=jnp.float32) * (1.0 / TEMP)


def _row_spec(w):
    return pl.BlockSpec((BLKR, w), lambda i: (i, 0))


def _full_spec(shape):
    nd = len(shape)
    return pl.BlockSpec(shape, lambda *_: (0,) * nd)


def _part_spec(w):
    return pl.BlockSpec((2, BLKR, w), lambda i: (0, i, 0))


def _tc1(x, q, w1):
    return pl.pallas_call(
        _t1_body,
        grid=(NBLK,),
        in_specs=[_row_spec(D), _full_spec((1, D)), _full_spec((D, H))],
        out_specs=[_row_spec(D), _row_spec(H)],
        out_shape=[jax.ShapeDtypeStruct((NP, D), jnp.float32),
                   jax.ShapeDtypeStruct((NP, H), jnp.float32)],
    )(x, q, w1)


def _tc2(degp, fea, z1):
    return pl.pallas_call(
        _t2_body,
        grid=(NBLK,),
        in_specs=[_part_spec(D), _row_spec(D), _row_spec(H)],
        out_specs=[_row_spec(D), _row_spec(D), _row_spec(H)],
        out_shape=[jax.ShapeDtypeStruct((NP, D), jnp.float32),
                   jax.ShapeDtypeStruct((NP, D), jnp.float32),
                   jax.ShapeDtypeStruct((NP, H), jnp.float32)],
    )(degp, fea, z1)


def _tc3(fea, dis, g, tmpp, pb):
    return pl.pallas_call(
        _t3_body,
        grid=(NBLK,),
        in_specs=[_row_spec(D), _row_spec(D), _row_spec(D), _part_spec(D),
                  _full_spec((1, 2 * D))],
        out_specs=_row_spec(2 * D),
        out_shape=jax.ShapeDtypeStruct((NP, 2 * D), jnp.float32),
    )(fea, dis, g, tmpp, pb)


def _tc4(hn, hnt):
    return pl.pallas_call(
        _t4_body,
        grid=(NBLK,),
        in_specs=[_row_spec(2 * D), _full_spec((2 * D, NP))],
        out_specs=[_row_spec(KP), _row_spec(D), _row_spec(KP)],
        out_shape=[jax.ShapeDtypeStruct((NP, KP), jnp.int32),
                   jax.ShapeDtypeStruct((NP, D), jnp.int32),
                   jax.ShapeDtypeStruct((NP, KP), jnp.float32)],
        scratch_shapes=[pltpu.VMEM((BLKR, NP), jnp.float32)],
    )(hn, hnt)


def _tc5(al, s1p, feat, dis, s2, s3p, w2):
    return pl.pallas_call(
        _t5_body,
        grid=(NBLK,),
        in_specs=[pl.BlockSpec(memory_space=pltpu.SMEM),
                  _part_spec(H), _row_spec(H), _row_spec(D), _row_spec(H),
                  _part_spec(H), _full_spec((H, H))],
        out_specs=[_row_spec(H), _row_spec(H)],
        out_shape=[jax.ShapeDtypeStruct((NP, H), jnp.float32),
                   jax.ShapeDtypeStruct((NP, H), jnp.float32)],
    )(al, s1p, feat, dis, s2, s3p, w2)


def _tc6a(al, s1p, feat, dis, s2, s3p):
    return pl.pallas_call(
        _t6a_body,
        grid=(NBLK,),
        in_specs=[pl.BlockSpec(memory_space=pltpu.SMEM),
                  _part_spec(H), _row_spec(H), _row_spec(D), _row_spec(H),
                  _part_spec(H)],
        out_specs=_row_spec(H),
        out_shape=jax.ShapeDtypeStruct((NP, H), jnp.float32),
    )(al, s1p, feat, dis, s2, s3p)


def _tc6b(out, nidx, lab):
    return pl.pallas_call(
        _t6b_body,
        in_specs=[_full_spec((NP, H)), _full_spec((NSEL, 1)),
                  _full_spec((NSEL, 1))],
        out_specs=_full_spec((NSEL, C)),
        out_shape=jax.ShapeDtypeStruct((NSEL, C), jnp.float32),
    )(out, nidx, lab)


_MESH = dict(core_axis_name="c", subcore_axis_name="s")
NSUB = 16
NTILE = 32
RPS = NP // NSUB
RPT = NP // NTILE
ECH = EP // (NTILE * 128)


def _edge_scatter(srcs2d, dsts2d, table, zrows):
    w = table.shape[1]

    @functools.partial(
        pl.kernel,
        out_type=jax.ShapeDtypeStruct((2, NP, w), jnp.float32),
        mesh=plsc.VectorSubcoreMesh(**_MESH),
        scratch_types=[
            pltpu.VMEM((ECH, 128), jnp.int32),
            pltpu.VMEM((ECH, 128), jnp.int32),
            pltpu.VMEM((128, w), jnp.float32),
            pltpu.VMEM_SHARED((NP, w), jnp.float32),
            pltpu.SemaphoreType.DMA,
        ],
    )
    def k(srcs_hbm, dsts_hbm, table_hbm, z_hbm, out_hbm, sidx, didx, rows,
          accum, sem):
        c = lax.axis_index("c")
        s = lax.axis_index("s")
        pltpu.sync_copy(z_hbm.at[pl.ds(s * RPS, RPS)],
                        accum.at[pl.ds(s * RPS, RPS)])
        plsc.subcore_barrier()
        base = (c * NSUB + s) * ECH
        pltpu.sync_copy(srcs_hbm.at[pl.ds(base, ECH)], sidx)
        pltpu.sync_copy(dsts_hbm.at[pl.ds(base, ECH)], didx)

        def step(j, carry):
            pltpu.async_copy(table_hbm.at[sidx.at[j]], rows, sem).wait()
            pltpu.sync_copy(rows, accum.at[didx.at[j]], add=True)
            return carry

        lax.fori_loop(0, ECH, step, 0)
        plsc.subcore_barrier()
        pltpu.sync_copy(accum.at[pl.ds(s * RPS, RPS)],
                        out_hbm.at[c, pl.ds(s * RPS, RPS)])

    return k(srcs2d, dsts2d, table, zrows)


def _gather_hits(nbr, nbrw):
    @functools.partial(
        pl.kernel,
        out_type=jax.ShapeDtypeStruct((NP * KP, 16), jnp.int32),
        mesh=plsc.VectorSubcoreMesh(**_MESH),
        scratch_types=[
            pltpu.VMEM((16, KP), jnp.int32),
            pltpu.VMEM((KP, D), jnp.int32),
            pltpu.VMEM((KP, 16), jnp.int32),
            pltpu.SemaphoreType.DMA,
        ],
    )
    def k(nbr_hbm, nbrw_hbm, hits_hbm, nbc, grows, hbuf, sem):
        c = lax.axis_index("c")
        s = lax.axis_index("s")
        tile = c * NSUB + s

        def chunk_fn(ch, carry):
            row0 = tile * RPT + ch * 16
            pltpu.sync_copy(nbr_hbm.at[pl.ds(row0, 16)], nbc)

            def row_fn(r, carry2):
                pltpu.async_copy(nbrw_hbm.at[nbc.at[r]], grows, sem).wait()
                rids = jnp.full((16,), row0 + r, jnp.int32)
                one = jnp.full((16,), 1, jnp.int32)
                for ee in range(KP):
                    h = jnp.zeros((16,), jnp.int32)
                    for m3 in range(KP // 16):
                        d = jnp.abs(grows[ee, pl.ds(m3 * 16, 16)] - rids)
                        h = jnp.maximum(h, one - jnp.minimum(d, one))
                    hbuf[ee, pl.ds(0, 16)] = h
                pltpu.sync_copy(
                    hbuf, hits_hbm.at[pl.ds((row0 + r) * KP, KP)])
                return carry2

            lax.fori_loop(0, 16, row_fn, 0)
            return carry

        lax.fori_loop(0, RPT // 16, chunk_fn, 0)

    return k(nbr, nbrw)


def _tw_body(hits_ref, vals_ref, weff_ref):
    pio = lax.broadcasted_iota(jnp.int32, (16 * KP, KP), 0)
    eio = lax.broadcasted_iota(jnp.int32, (16 * KP, KP), 1)
    gmat = (pio // 16 == eio).astype(jnp.float32)
    cnt = jnp.dot(hits_ref[...].astype(jnp.float32), gmat,
                  preferred_element_type=jnp.float32)
    weff_ref[...] = jnp.maximum(vals_ref[...], 0.0) * jnp.where(
        cnt > 0.0, 0.5, 1.0)


def _tc_weff(hitsr, vals):
    return pl.pallas_call(
        _tw_body,
        grid=(NBLK,),
        in_specs=[_row_spec(16 * KP), _row_spec(KP)],
        out_specs=_row_spec(KP),
        out_shape=jax.ShapeDtypeStruct((NP, KP), jnp.float32),
    )(hitsr, vals)


def _knn_prop(nbr, weff, feat, zrows):
    @functools.partial(
        pl.kernel,
        out_type=[jax.ShapeDtypeStruct((NP, H), jnp.float32),
                  jax.ShapeDtypeStruct((2, NP, H), jnp.float32)],
        mesh=plsc.VectorSubcoreMesh(**_MESH),
        scratch_types=[
            pltpu.VMEM((16, KP), jnp.int32),
            pltpu.VMEM((16, KP), jnp.float32),
            pltpu.VMEM((16, H), jnp.float32),
            pltpu.VMEM((KP, H), jnp.float32),
            pltpu.VMEM((KP, H), jnp.float32),
            pltpu.VMEM((16, H), jnp.float32),
            pltpu.VMEM_SHARED((NP, H), jnp.float32),
            pltpu.SemaphoreType.DMA,
        ],
    )
    def k(nbr_hbm, w_hbm, feat_hbm, z_hbm, s2_hbm, out_hbm,
          nbc, wbc, fic, fbuf, sbuf, s2c, accum, sem):
        c = lax.axis_index("c")
        s = lax.axis_index("s")
        tile = c * NSUB + s
        pltpu.sync_copy(z_hbm.at[pl.ds(s * RPS, RPS)],
                        accum.at[pl.ds(s * RPS, RPS)])
        plsc.subcore_barrier()

        def chunk_fn(ch, carry):
            row0 = tile * RPT + ch * 16
            pltpu.sync_copy(nbr_hbm.at[pl.ds(row0, 16)], nbc)
            pltpu.sync_copy(w_hbm.at[pl.ds(row0, 16)], wbc)
            pltpu.sync_copy(feat_hbm.at[pl.ds(row0, 16)], fic)

            def row_fn(r, carry2):
                pltpu.async_copy(feat_hbm.at[nbc.at[r]], fbuf, sem).wait()
                fi = [fic[r, pl.ds(c8 * 16, 16)] for c8 in range(H // 16)]
                acc = [jnp.zeros((16,), jnp.float32) for _ in range(H // 16)]
                for g in range(KP // 16):
                    wv = wbc[r, pl.ds(g * 16, 16)]
                    for e in range(16):
                        kk = g * 16 + e
                        wk = jnp.full((16,), wv[e], jnp.float32)
                        for c8 in range(H // 16):
                            fv = fbuf[kk, pl.ds(c8 * 16, 16)]
                            acc[c8] = acc[c8] + wk * fv
                            sbuf[kk, pl.ds(c8 * 16, 16)] = wk * fi[c8]
                for c8 in range(H // 16):
                    s2c[r, pl.ds(c8 * 16, 16)] = acc[c8]
                pltpu.sync_copy(sbuf, accum.at[nbc.at[r]], add=True)
                return carry2

            lax.fori_loop(0, 16, row_fn, 0)
            pltpu.sync_copy(s2c, s2_hbm.at[pl.ds(row0, 16)])
            return carry

        lax.fori_loop(0, RPT // 16, chunk_fn, 0)
        plsc.subcore_barrier()
        pltpu.sync_copy(accum.at[pl.ds(s * RPS, RPS)],
                        out_hbm.at[c, pl.ds(s * RPS, RPS)])

    return k(nbr, weff, feat, zrows)


def kernel(x, edge_index, node_idx, labels, p_hol, p_shared, combine_weight,
           p_balance, W1, W2, alpha):
    f32 = jnp.float32
    xp = jnp.concatenate([x, jnp.zeros((NP - N0, D), f32)])
    src = jnp.concatenate(
        [edge_index[0], jnp.full((EP - E0,), ND, jnp.int32)]).reshape(-1, 128)
    dst = jnp.concatenate(
        [edge_index[1], jnp.full((EP - E0,), ND, jnp.int32)]).reshape(-1, 128)
    q = (combine_weight[0, 0] * p_hol
         + combine_weight[0, 1] * p_shared).reshape(1, D)
    pb = p_balance.reshape(1, 2 * D)
    al = alpha.reshape(1, 1)

    fea, z1 = _tc1(xp, q, W1)
    ones128 = jnp.ones((NP, D), f32)
    z128 = jnp.zeros((NP, D), f32)
    degp = _edge_scatter(dst, dst, ones128, z128)
    dis, g, t1 = _tc2(degp, fea, z1)
    tmpp = _edge_scatter(src, dst, g, z128)
    hn = _tc3(fea, dis, g, tmpp, pb)
    nbr, nbrw, vals = _tc4(hn, hn.T)

    hits = _gather_hits(nbr, nbrw)
    weff = _tc_weff(hits.reshape(NP, KP * 16), vals)

    s1p1 = _edge_scatter(src, dst, t1, z128)
    s2_1, s3p1 = _knn_prop(nbr, weff, z1, z128)
    z2, t2 = _tc5(al, s1p1, z1, dis, s2_1, s3p1, W2)

    s1p2 = _edge_scatter(src, dst, t2, z128)
    s2_2, s3p2 = _knn_prop(nbr, weff, z2, z128)
    out = _tc6a(al, s1p2, z2, dis, s2_2, s3p2)

    return _tc6b(out, node_idx.reshape(NSEL, 1), labels.reshape(NSEL, 1))
